# baseline (speedup 1.0000x reference)
import jax
import jax.numpy as jnp
from jax import lax
from jax.experimental import pallas as pl
from jax.experimental.pallas import tpu as pltpu
from jax.experimental.pallas import tpu_sc as plsc

_BATCH = 4
_SEQ = 8192
_D = 768
_NC = 2
_NS = 16
_NW = _NC * _NS
_ROWS_PER_W = _SEQ // _NW
_CHUNK = 64
_NCHUNK = _ROWS_PER_W // _CHUNK


def _sc_body(pe_hbm, out_hbm, buf0, buf1, isem0, isem1, osem0, osem1):
    wid = lax.axis_index("s") * _NC + lax.axis_index("c")
    base = wid * _ROWS_PER_W
    bufs = (buf0, buf1)
    isems = (isem0, isem1)
    osems = (osem0, osem1)

    def in_copy(k):
        return pltpu.make_async_copy(
            pe_hbm.at[pl.ds(base + k * _CHUNK, _CHUNK), :],
            bufs[k % 2],
            isems[k % 2],
        )

    def out_copies(k):
        return [
            pltpu.make_async_copy(
                bufs[k % 2],
                out_hbm.at[b, pl.ds(base + k * _CHUNK, _CHUNK), :],
                osems[k % 2],
            )
            for b in range(_BATCH)
        ]

    in_copy(0).start()
    in_copy(1).start()
    for k in range(_NCHUNK):
        if k >= 2:
            for cp in out_copies(k - 2):
                cp.wait()
            in_copy(k).start()
        in_copy(k).wait()
        for cp in out_copies(k):
            cp.start()
    for k in (_NCHUNK - 2, _NCHUNK - 1):
        for cp in out_copies(k):
            cp.wait()


def kernel(input_ids, pos_embed):
    batch, seq_len = input_ids.shape
    d = pos_embed.shape[1]
    mesh = plsc.VectorSubcoreMesh(core_axis_name="c", subcore_axis_name="s")
    sc_call = pl.kernel(
        _sc_body,
        out_type=jax.ShapeDtypeStruct((batch, seq_len, d), jnp.float32),
        mesh=mesh,
        scratch_types=[
            pltpu.VMEM((_CHUNK, _D), jnp.float32),
            pltpu.VMEM((_CHUNK, _D), jnp.float32),
            pltpu.SemaphoreType.DMA,
            pltpu.SemaphoreType.DMA,
            pltpu.SemaphoreType.DMA,
            pltpu.SemaphoreType.DMA,
        ],
    )
    return sc_call(pos_embed[:seq_len])


# speedup vs baseline: 1.0045x; 1.0045x over previous
import jax
import jax.numpy as jnp
from jax import lax
from jax.experimental import pallas as pl
from jax.experimental.pallas import tpu as pltpu
from jax.experimental.pallas import tpu_sc as plsc

_BATCH = 4
_SEQ = 8192
_D = 768

_NC = 2
_NS = 16
_NW = _NC * _NS
_SC_ROWS = 1024
_ROWS_PER_W = _SC_ROWS // _NW
_CHUNK = _ROWS_PER_W // 2
_NCHUNK = 2

_TC_BS = 1024
_TC_OFF = _SC_ROWS // _TC_BS
_TC_GRID = (_SEQ - _SC_ROWS) // _TC_BS


def _sc_body(pe_hbm, out_hbm, buf0, buf1, isem0, isem1, osem0, osem1):
    wid = lax.axis_index("s") * _NC + lax.axis_index("c")
    base = wid * _ROWS_PER_W
    bufs = (buf0, buf1)
    isems = (isem0, isem1)
    osems = (osem0, osem1)

    def in_copy(k):
        return pltpu.make_async_copy(
            pe_hbm.at[pl.ds(base + k * _CHUNK, _CHUNK), :],
            bufs[k % 2],
            isems[k % 2],
        )

    def out_copies(k):
        return [
            pltpu.make_async_copy(
                bufs[k % 2],
                out_hbm.at[b, pl.ds(base + k * _CHUNK, _CHUNK), :],
                osems[k % 2],
            )
            for b in range(_BATCH)
        ]

    in_copy(0).start()
    in_copy(1).start()
    for k in range(_NCHUNK):
        in_copy(k).wait()
        for cp in out_copies(k):
            cp.start()
    for k in range(_NCHUNK):
        for cp in out_copies(k):
            cp.wait()


def _tc_body(partial_ref, pe_ref, out_ref):
    del partial_ref
    out_ref[...] = jnp.broadcast_to(pe_ref[...][None, :, :], out_ref.shape)


def kernel(input_ids, pos_embed):
    batch, seq_len = input_ids.shape
    d = pos_embed.shape[1]
    sc_mesh = plsc.VectorSubcoreMesh(core_axis_name="c", subcore_axis_name="s")
    sc_call = pl.kernel(
        _sc_body,
        out_type=jax.ShapeDtypeStruct((batch, seq_len, d), jnp.float32),
        mesh=sc_mesh,
        scratch_types=[
            pltpu.VMEM((_CHUNK, _D), jnp.float32),
            pltpu.VMEM((_CHUNK, _D), jnp.float32),
            pltpu.SemaphoreType.DMA,
            pltpu.SemaphoreType.DMA,
            pltpu.SemaphoreType.DMA,
            pltpu.SemaphoreType.DMA,
        ],
    )
    partial = sc_call(pos_embed[:seq_len])

    return pl.pallas_call(
        _tc_body,
        grid=(_TC_GRID,),
        in_specs=[
            pl.BlockSpec(memory_space=pl.ANY),
            pl.BlockSpec((_TC_BS, d), lambda i: (i + _TC_OFF, 0)),
        ],
        out_specs=pl.BlockSpec((batch, _TC_BS, d), lambda i: (0, i + _TC_OFF, 0)),
        out_shape=jax.ShapeDtypeStruct((batch, seq_len, d), jnp.float32),
        input_output_aliases={0: 0},
        compiler_params=pltpu.CompilerParams(
            dimension_semantics=("arbitrary",),
        ),
    )(partial, pos_embed[:seq_len])


# speedup vs baseline: 1.3323x; 1.3263x over previous
import jax
import jax.numpy as jnp
from jax.experimental import pallas as pl
from jax.experimental.pallas import tpu as pltpu

_BATCH = 4
_BS = 2048


def _body(pe_ref, out_ref):
    out_ref[...] = pe_ref[...][None, :, :]


def kernel(input_ids, pos_embed):
    batch, seq_len = input_ids.shape
    d = pos_embed.shape[1]
    grid = (seq_len // _BS, batch)
    return pl.pallas_call(
        _body,
        grid=grid,
        in_specs=[pl.BlockSpec((_BS, d), lambda i, j: (i, 0))],
        out_specs=pl.BlockSpec((1, _BS, d), lambda i, j: (j, i, 0)),
        out_shape=jax.ShapeDtypeStruct((batch, seq_len, d), jnp.float32),
        compiler_params=pltpu.CompilerParams(
            dimension_semantics=("arbitrary", "arbitrary"),
        ),
    )(pos_embed[:seq_len])
